# Initial kernel scaffold; baseline (speedup 1.0000x reference)
import jax
import jax.numpy as jnp
from jax.experimental import pallas as pl


def kernel(x_q, x_s, x_z, centroids_q, centroids_s, centroids_z, lut_q, lut_s, lut_z, bias_q, bias_s, bias_z, output_s, output_z):
    raise NotImplementedError("write your pallas kernel here")


# speedup vs baseline: 20.1529x; 20.1529x over previous
import functools

import jax
import jax.numpy as jnp
import numpy as np
from jax.experimental import pallas as pl
from jax.experimental.pallas import tpu as pltpu

CIN = 96
COUT = 96
NCB = 16
K = 16
SUB = 54
CPB = CIN // NCB
OH = 112
OW = 112
TH = 8
NT = OH // TH
NTILE = TH * OW
NL = NCB * K


def _body(params_ref, x_ref, cblk_ref, t2k_ref, den_ref, lut_ref, bias_ref,
          out_ref):
    t = pl.program_id(1)
    oy0 = t * TH
    x_sc = params_ref[3]
    x_zp = params_ref[4]

    s_acc = jnp.zeros((NTILE, NL), jnp.float32)
    for ki in range(3):
        pr, wr = (ki + 1) // 2, (ki + 1) % 2
        for kj in range(3):
            pc, wc = (kj + 1) // 2, (kj + 1) % 2
            xs8 = x_ref[0, wr, wc, pl.ds(oy0 + pr, TH), pl.ds(pc, OW), :]
            xdq = (xs8.reshape(NTILE, CIN).astype(jnp.float32) - x_zp) * x_sc
            xbf = xdq.astype(jnp.bfloat16)
            cp = cblk_ref[(ki * 3 + kj) * CIN:(ki * 3 + kj + 1) * CIN, :]
            s_acc = s_acc + jax.lax.dot_general(
                xbf, cp, (((1,), (0,)), ((), ())),
                preferred_element_type=jnp.float32)

    r = jnp.round(s_acc / den_ref[0:1, :])
    key = t2k_ref[0:1, :] - 32 * r.astype(jnp.int32)

    lane = jax.lax.broadcasted_iota(jnp.int32, (NTILE, NL), 1)
    m = key
    for sh in (1, 2, 4, 8):
        up = jnp.roll(m, -sh, axis=1)
        dn = jnp.roll(m, sh, axis=1)
        m = jnp.minimum(m, jnp.where((lane & sh) == 0, up, dn))

    onehot = (key == m).astype(jnp.bfloat16)
    acc = jax.lax.dot_general(
        onehot, lut_ref[...], (((1,), (0,)), ((), ())),
        preferred_element_type=jnp.float32)

    lut_s = params_ref[0]
    out_s = params_ref[1]
    out_z = params_ref[2]
    outf = acc * lut_s + bias_ref[0:1, :]
    outf = jnp.maximum(outf, 0.0)
    q = jnp.clip(outf / out_s + out_z, -128.0, 127.0)
    out_ref[...] = jnp.round(q).astype(jnp.int8).reshape(1, TH, OW, COUT)


@functools.partial(jax.jit, static_argnames=())
def kernel(x_q, x_s, x_z, centroids_q, centroids_s, centroids_z,
           lut_q, lut_s, lut_z, bias_q, bias_s, bias_z, output_s, output_z):
    b = x_q.shape[0]

    x8 = x_q.astype(jnp.int8)
    xt = jnp.transpose(x8, (0, 2, 3, 1))
    xp = jnp.pad(xt, ((0, 0), (2, 0), (2, 0), (0, 0)))
    xr = xp.reshape(b, 113, 2, 113, 2, CIN)
    xprep = jnp.transpose(xr, (0, 2, 4, 1, 3, 5))

    cq = (centroids_q - centroids_z).astype(jnp.float32)
    cbi = np.arange(NCB)[:, None, None]
    kii = np.arange(K)[None, :, None]
    sii = np.arange(SUB)[None, None, :]
    dprime = (sii % 9) * CIN + (cbi * CPB + sii // 9)
    col = cbi * K + kii
    dprime = np.broadcast_to(dprime, (NCB, K, SUB))
    col = np.broadcast_to(col, (NCB, K, SUB))
    c_dq = cq * centroids_s
    cblk = jnp.zeros((9 * CIN, NL), jnp.float32).at[dprime, col].set(c_dq)
    cblk_bf = cblk.astype(jnp.bfloat16)

    y2 = jnp.sum(c_dq * c_dq, axis=-1)
    den = x_s[0] * centroids_s[:, 0, 0]
    t2 = jnp.round(y2 / den[:, None])
    t2i = t2.astype(jnp.int32)
    t2k = (16 * t2i + jnp.arange(K, dtype=jnp.int32)[None, :]).reshape(1, NL)
    den_l = jnp.broadcast_to(den[:, None], (NCB, K)).reshape(1, NL)

    lut_bf = (lut_q - lut_z[0]).astype(jnp.bfloat16).reshape(NL, COUT)
    bias_f = ((bias_q - bias_z[0]).astype(jnp.float32) * bias_s[0]
              ).reshape(1, COUT)
    params = jnp.stack([lut_s[0], output_s[0],
                        output_z[0].astype(jnp.float32), x_s[0],
                        x_z[0].astype(jnp.float32)])

    out_nhwc = pl.pallas_call(
        _body,
        grid=(b, NT),
        in_specs=[
            pl.BlockSpec(memory_space=pltpu.SMEM),
            pl.BlockSpec((1, 2, 2, 113, 113, CIN),
                         lambda bb, tt: (bb, 0, 0, 0, 0, 0)),
            pl.BlockSpec((9 * CIN, NL), lambda bb, tt: (0, 0)),
            pl.BlockSpec((1, NL), lambda bb, tt: (0, 0)),
            pl.BlockSpec((1, NL), lambda bb, tt: (0, 0)),
            pl.BlockSpec((NL, COUT), lambda bb, tt: (0, 0)),
            pl.BlockSpec((1, COUT), lambda bb, tt: (0, 0)),
        ],
        out_specs=pl.BlockSpec((1, TH, OW, COUT),
                               lambda bb, tt: (bb, tt, 0, 0)),
        out_shape=jax.ShapeDtypeStruct((b, OH, OW, COUT), jnp.int8),
        compiler_params=pltpu.CompilerParams(
            dimension_semantics=("arbitrary", "arbitrary")),
    )(params, xprep, cblk_bf, t2k, den_l, lut_bf, bias_f)

    return jnp.transpose(out_nhwc, (0, 3, 1, 2))


# speedup vs baseline: 30.5507x; 1.5159x over previous
import functools

import jax
import jax.numpy as jnp
import numpy as np
from jax.experimental import pallas as pl
from jax.experimental.pallas import tpu as pltpu

CIN = 96
COUT = 96
NCB = 16
K = 16
SUB = 54
CPB = CIN // NCB
OH = 112
OW = 112
NL = NCB * K
TH = 8
NT = OH // TH
NW = TH * OW


def _body(params_ref, xc_ref, xp_ref, dd_ref, cblkt_ref, t2k_ref, den_ref,
          lutt_ref, bias_ref, out_ref):
    t = pl.program_id(1)
    x_sc = params_ref[3]
    x_zp = params_ref[4]

    eo = []
    for l in range(17):
        if l == 0:
            r32 = xp_ref[0, :, 7, :]
            w = jnp.where(t > 0, x_sc, 0.0)
        else:
            r32 = xc_ref[0, :, l - 1, :]
            w = x_sc
        rbf = ((r32.astype(jnp.float32) - x_zp) * w).astype(jnp.bfloat16)
        eo.append(jax.lax.dot_general(
            rbf, dd_ref[...], (((1,), (0,)), ((), ())),
            preferred_element_type=jnp.float32).astype(jnp.bfloat16))

    zcol = jnp.zeros((CIN, 1), jnp.bfloat16)
    s_acc = jnp.zeros((NL, NW), jnp.float32)
    for ki in range(3):
        rows = [eo[2 * rr + ki] for rr in range(TH)]
        ev = jnp.concatenate([e[:, :OW] for e in rows], axis=1)
        od = jnp.concatenate([e[:, OW:2 * OW] for e in rows], axis=1)
        odm = jnp.concatenate(
            sum(([zcol, e[:, OW:2 * OW - 1]] for e in rows), []), axis=1)
        for kj, tap in ((0, odm), (1, ev), (2, od)):
            cp = cblkt_ref[:, (ki * 3 + kj) * CIN:(ki * 3 + kj + 1) * CIN]
            s_acc = s_acc + jax.lax.dot_general(
                cp, tap, (((1,), (0,)), ((), ())),
                preferred_element_type=jnp.float32)

    r = jnp.round(s_acc / den_ref[:, 0:1])
    key = t2k_ref[:, 0:1] - 32 * r.astype(jnp.int32)

    sub = jax.lax.broadcasted_iota(jnp.int32, (NL, NW), 0)
    m = key
    for sh in (1, 2, 4, 8):
        up = jnp.roll(m, -sh, axis=0)
        dn = jnp.roll(m, sh, axis=0)
        m = jnp.minimum(m, jnp.where((sub & sh) == 0, up, dn))

    onehot = (key == m).astype(jnp.bfloat16)
    acc = jax.lax.dot_general(
        lutt_ref[...], onehot, (((1,), (0,)), ((), ())),
        preferred_element_type=jnp.float32)

    lut_s = params_ref[0]
    out_s = params_ref[1]
    out_z = params_ref[2]
    outf = acc * lut_s + bias_ref[:, 0:1]
    outf = jnp.maximum(outf, 0.0)
    q = jnp.clip(outf / out_s + out_z, -128.0, 127.0)
    q8 = jnp.round(q).astype(jnp.int8)
    for rr in range(TH):
        out_ref[0, :, rr, 0, :] = q8[:, rr * OW:(rr + 1) * OW]


@functools.partial(jax.jit, static_argnames=())
def kernel(x_q, x_s, x_z, centroids_q, centroids_s, centroids_z,
           lut_q, lut_s, lut_z, bias_q, bias_s, bias_z, output_s, output_z):
    b = x_q.shape[0]

    dd = np.zeros((224, 224), np.float32)
    dd[np.arange(0, 224, 2), np.arange(112)] = 1.0
    dd[np.arange(1, 224, 2), np.arange(112, 224)] = 1.0
    dd_bf = jnp.asarray(dd, dtype=jnp.bfloat16)

    cq = (centroids_q - centroids_z).astype(jnp.float32)
    c_dq = cq * centroids_s
    cbi = np.arange(NCB)[:, None, None]
    kii = np.arange(K)[None, :, None]
    sii = np.arange(SUB)[None, None, :]
    dprime = (sii % 9) * CIN + (cbi * CPB + sii // 9)
    col = cbi * K + kii
    dprime = np.broadcast_to(dprime, (NCB, K, SUB))
    col = np.broadcast_to(col, (NCB, K, SUB))
    cblkt = jnp.zeros((NL, 9 * CIN), jnp.float32).at[col, dprime].set(c_dq)
    cblkt_bf = cblkt.astype(jnp.bfloat16)

    y2 = jnp.sum(c_dq * c_dq, axis=-1)
    den = x_s[0] * centroids_s[:, 0, 0]
    t2 = jnp.round(y2 / den[:, None])
    t2k = (16 * t2.astype(jnp.int32)
           + jnp.arange(K, dtype=jnp.int32)[None, :]).reshape(NL, 1)
    den_c = jnp.broadcast_to(den[:, None], (NCB, K)).reshape(NL, 1)

    lutt_bf = (lut_q - lut_z[0]).astype(jnp.bfloat16).reshape(NL, COUT).T
    bias_f = ((bias_q - bias_z[0]).astype(jnp.float32) * bias_s[0]
              ).reshape(COUT, 1)
    params = jnp.stack([lut_s[0], output_s[0],
                        output_z[0].astype(jnp.float32), x_s[0],
                        x_z[0].astype(jnp.float32)])

    out5 = pl.pallas_call(
        _body,
        grid=(b, NT),
        in_specs=[
            pl.BlockSpec(memory_space=pltpu.SMEM),
            pl.BlockSpec((1, CIN, 16, 224), lambda bb, tt: (bb, 0, tt, 0)),
            pl.BlockSpec((1, CIN, 8, 224),
                         lambda bb, tt: (bb, 0, jnp.maximum(2 * tt - 1, 0), 0)),
            pl.BlockSpec((224, 224), lambda bb, tt: (0, 0)),
            pl.BlockSpec((NL, 9 * CIN), lambda bb, tt: (0, 0)),
            pl.BlockSpec((NL, 1), lambda bb, tt: (0, 0)),
            pl.BlockSpec((NL, 1), lambda bb, tt: (0, 0)),
            pl.BlockSpec((COUT, NL), lambda bb, tt: (0, 0)),
            pl.BlockSpec((COUT, 1), lambda bb, tt: (0, 0)),
        ],
        out_specs=pl.BlockSpec((1, COUT, TH, 1, OW),
                               lambda bb, tt: (bb, 0, tt, 0, 0)),
        out_shape=jax.ShapeDtypeStruct((b, COUT, OH, 1, OW), jnp.int8),
        compiler_params=pltpu.CompilerParams(
            dimension_semantics=("arbitrary", "arbitrary")),
    )(params, x_q, x_q, dd_bf, cblkt_bf, t2k, den_c, lutt_bf, bias_f)
    return out5.reshape(b, COUT, OH, OW)
